# baseline (speedup 1.0000x reference)
import functools

import jax
import jax.numpy as jnp
from jax import lax
from jax.experimental import pallas as pl
from jax.experimental.pallas import tpu as pltpu
from jax.experimental.pallas import tpu_sc as plsc

EMBED = 64
QUAD = 4
QROWS = 96
QCOL = QUAD * EMBED
GROUP = 128
CHUNK = GROUP * QUAD
NBUF = 3


def _quad_table_body(t_ref, o_ref):
    t = t_ref[...]
    r = lax.broadcasted_iota(jnp.int32, (QROWS, EMBED), 0)
    rows = [jnp.broadcast_to(t[k:k + 1, :], (QROWS, EMBED)) for k in range(3)]
    parts = []
    for k in range(QUAD):
        digit = (r // (3 ** (QUAD - 1 - k))) % 3
        parts.append(jnp.where(digit == 0, rows[0],
                               jnp.where(digit == 1, rows[1], rows[2])))
    o_ref[...] = jnp.concatenate(parts, axis=1)


NREP = 32


def _make_quad_table(seg_table):
    return pl.pallas_call(
        _quad_table_body,
        grid=(NREP,),
        in_specs=[pl.BlockSpec((3, EMBED), lambda r: (0, 0))],
        out_specs=pl.BlockSpec((QROWS, QCOL), lambda r: (r, 0)),
        out_shape=jax.ShapeDtypeStruct((NREP * QROWS, QCOL), jnp.float32),
    )(seg_table)


@functools.cache
def _make_sc_lookup(B: int):
    info = plsc.get_sparse_core_info()
    nw = info.num_cores * info.num_subcores
    b_per_w = B // nw
    assert B % nw == 0 and b_per_w % CHUNK == 0
    assert (b_per_w // CHUNK - 2) % NBUF == 0 and b_per_w // CHUNK >= 5
    mesh = plsc.VectorSubcoreMesh(core_axis_name="c", subcore_axis_name="s")

    @functools.partial(
        pl.kernel,
        mesh=mesh,
        out_type=jax.ShapeDtypeStruct((B // QUAD, QCOL), jnp.float32),
        scratch_types=[
            [pltpu.VMEM((GROUP,), jnp.int32) for _ in range(NBUF)],
            [pltpu.VMEM((GROUP,), jnp.int32) for _ in range(NBUF)],
            [pltpu.VMEM((GROUP, QCOL), jnp.float32) for _ in range(NBUF)],
            [pltpu.SemaphoreType.DMA for _ in range(NBUF)],
            [pltpu.SemaphoreType.DMA for _ in range(NBUF)],
            [pltpu.SemaphoreType.DMA for _ in range(NBUF)],
        ],
    )
    def lookup(x_hbm, qt_hbm, out_hbm, idx_v, pidx_v, rows_v,
               sem_a, sem_g, sem_w):
        wid = lax.axis_index("s") * info.num_cores + lax.axis_index("c")
        baseq = wid * (b_per_w // QUAD)
        n_chunks = b_per_w // CHUNK

        def offq(g):
            return pl.multiple_of(baseq + g * GROUP, GROUP)

        def issue_a(g, b):
            pltpu.async_copy(x_hbm.at[pl.ds(offq(g), GROUP)],
                             idx_v[b], sem_a[b])

        def wait_a(b):
            pltpu.make_async_copy(x_hbm.at[pl.ds(0, GROUP)],
                                  idx_v[b], sem_a[b]).wait()

        rep_off = wid * QROWS

        def pack(b):
            for j in range(GROUP // 16):
                pidx_v[b][pl.ds(j * 16, 16)] = \
                    idx_v[b][pl.ds(j * 16, 16)] + rep_off

        def issue_g(b):
            pltpu.async_copy(qt_hbm.at[pidx_v[b]], rows_v[b], sem_g[b])

        def wait_g(b):
            pltpu.make_async_copy(qt_hbm.at[pidx_v[b]],
                                  rows_v[b], sem_g[b]).wait()

        def issue_w(g, b):
            pltpu.async_copy(rows_v[b],
                             out_hbm.at[pl.ds(offq(g), GROUP)], sem_w[b])

        def wait_w(b):
            pltpu.make_async_copy(rows_v[b],
                                  out_hbm.at[pl.ds(0, GROUP)],
                                  sem_w[b]).wait()

        issue_a(0, 0)
        wait_a(0)
        pack(0)
        issue_g(0)
        issue_a(1, 1)
        wait_a(1)
        pack(1)
        issue_g(1)
        issue_a(2, 2)
        wait_a(2)
        pack(2)
        issue_g(2)
        issue_a(3, 0)
        wait_g(0)
        issue_w(0, 0)

        def outer(i, carry):
            for b in range(NBUF):
                g = i * NBUF + b
                wait_a(b)
                pack(b)
                wait_w(b)
                issue_g(b)
                issue_a(g + 1, (b + 1) % NBUF)
                bw = (b + 1) % NBUF
                wait_g(bw)
                issue_w(g - 2, bw)
            return carry

        lax.fori_loop(1, (n_chunks - 2) // NBUF, outer, 0)

        n = n_chunks
        wait_a(0)
        pack(0)
        wait_w(0)
        issue_g(0)
        issue_a(n - 1, 1)
        wait_g(1)
        issue_w(n - 4, 1)

        wait_a(1)
        pack(1)
        wait_w(1)
        issue_g(1)
        wait_g(2)
        issue_w(n - 3, 2)

        wait_g(0)
        issue_w(n - 2, 0)
        wait_g(1)
        issue_w(n - 1, 1)
        wait_w(2)
        wait_w(0)
        wait_w(1)

    return lookup


@functools.cache
def _make_sc_lookup_direct(R: int, C: int):
    IRC = 4
    QPR = C // QUAD
    QPC = IRC * QPR
    info = plsc.get_sparse_core_info()
    nw = info.num_cores * info.num_subcores
    rows_per_w = R // nw
    assert R % nw == 0 and rows_per_w % IRC == 0 and C % QUAD == 0
    n_chunks = rows_per_w // IRC
    mesh = plsc.VectorSubcoreMesh(core_axis_name="c", subcore_axis_name="s")

    @functools.partial(
        pl.kernel,
        mesh=mesh,
        out_type=jax.ShapeDtypeStruct((R, C, EMBED), jnp.float32),
        scratch_types=[
            pltpu.VMEM((112,), jnp.int32),
            pltpu.VMEM((96,), jnp.int32),
            pltpu.VMEM((104, QCOL), jnp.float32),
            pltpu.VMEM((IRC, C, EMBED), jnp.float32),
            pltpu.SemaphoreType.DMA,
        ],
    )
    def lookup(x_hbm, qt_hbm, out_hbm, idx_a, idx_b, gb, wb, sem):
        wid = lax.axis_index("s") * info.num_cores + lax.axis_index("c")
        rep_off = wid * QROWS
        row0 = wid * rows_per_w

        def retile_range(lo, cnt):
            def retile(q, c2):
                ii = q // QPR
                rr = (q % QPR) * QUAD
                for m in range(QUAD):
                    for t in range(EMBED // 16):
                        wb[ii, rr + m, pl.ds(t * 16, 16)] = \
                            gb[q - lo, pl.ds(m * EMBED + t * 16, 16)]
                return c2
            lax.fori_loop(lo, lo + cnt, retile, 0)

        def do_chunk(g, carry):
            i0 = row0 + g * IRC
            qoff = pl.multiple_of(i0 * QPR, QPC)
            pltpu.sync_copy(x_hbm.at[pl.ds(qoff, 104)],
                            idx_a.at[pl.ds(0, 104)])
            pltpu.sync_copy(x_hbm.at[pl.ds(qoff + 104, 96)], idx_b)
            for j in range(7):
                idx_a[pl.ds(j * 16, 16)] = \
                    idx_a[pl.ds(j * 16, 16)] + rep_off
            for j in range(6):
                idx_b[pl.ds(j * 16, 16)] = \
                    idx_b[pl.ds(j * 16, 16)] + rep_off
            pltpu.async_copy(qt_hbm.at[idx_a.at[pl.ds(0, 104)]],
                             gb.at[pl.ds(0, 104)], sem).wait()
            retile_range(0, 104)
            pltpu.async_copy(qt_hbm.at[idx_b],
                             gb.at[pl.ds(0, 96)], sem).wait()
            retile_range(104, 96)
            pltpu.sync_copy(wb, out_hbm.at[pl.ds(i0, IRC)])
            return carry

        lax.fori_loop(0, n_chunks, do_chunk, 0)

    return lookup


def kernel(x, seg_table):
    r, c = x.shape
    B = r * c
    xi = x.astype(jnp.int32)
    quads = (xi[:, 0::4] * 27 + xi[:, 1::4] * 9
             + xi[:, 2::4] * 3 + xi[:, 3::4])
    xb = quads.reshape(B // QUAD)
    qt = _make_quad_table(seg_table)
    return _make_sc_lookup_direct(r, c)(xb, qt)


# speedup vs baseline: 1.3221x; 1.3221x over previous
import functools

import jax
import jax.numpy as jnp
from jax import lax
from jax.experimental import pallas as pl
from jax.experimental.pallas import tpu as pltpu
from jax.experimental.pallas import tpu_sc as plsc

EMBED = 64
QUAD = 4
QROWS = 96
QCOL = QUAD * EMBED
GROUP = 128
CHUNK = GROUP * QUAD
NBUF = 3


def _quad_table_body(t_ref, o_ref):
    t = t_ref[...]
    r = lax.broadcasted_iota(jnp.int32, (QROWS, EMBED), 0)
    rows = [jnp.broadcast_to(t[k:k + 1, :], (QROWS, EMBED)) for k in range(3)]
    parts = []
    for k in range(QUAD):
        digit = (r // (3 ** (QUAD - 1 - k))) % 3
        parts.append(jnp.where(digit == 0, rows[0],
                               jnp.where(digit == 1, rows[1], rows[2])))
    o_ref[...] = jnp.concatenate(parts, axis=1)


NREP = 64


def _make_quad_table(seg_table):
    return pl.pallas_call(
        _quad_table_body,
        grid=(NREP,),
        in_specs=[pl.BlockSpec((3, EMBED), lambda r: (0, 0))],
        out_specs=pl.BlockSpec((QROWS, QCOL), lambda r: (r, 0)),
        out_shape=jax.ShapeDtypeStruct((NREP * QROWS, QCOL), jnp.float32),
    )(seg_table)


@functools.cache
def _make_sc_lookup(B: int):
    info = plsc.get_sparse_core_info()
    nw = info.num_cores * info.num_subcores
    b_per_w = B // nw
    assert B % nw == 0 and b_per_w % CHUNK == 0
    assert (b_per_w // CHUNK - 2) % NBUF == 0 and b_per_w // CHUNK >= 5
    mesh = plsc.VectorSubcoreMesh(core_axis_name="c", subcore_axis_name="s")

    @functools.partial(
        pl.kernel,
        mesh=mesh,
        out_type=jax.ShapeDtypeStruct((B // QUAD, QCOL), jnp.float32),
        scratch_types=[
            [pltpu.VMEM((GROUP,), jnp.int32) for _ in range(NBUF)],
            [pltpu.VMEM((GROUP,), jnp.int32) for _ in range(NBUF)],
            [pltpu.VMEM((GROUP, QCOL), jnp.float32) for _ in range(NBUF)],
            [pltpu.SemaphoreType.DMA for _ in range(NBUF)],
            [pltpu.SemaphoreType.DMA for _ in range(NBUF)],
            [pltpu.SemaphoreType.DMA for _ in range(NBUF)],
        ],
    )
    def lookup(x_hbm, qt_hbm, out_hbm, idx_v, pidx_v, rows_v,
               sem_a, sem_g, sem_w):
        wid = lax.axis_index("s") * info.num_cores + lax.axis_index("c")
        baseq = wid * (b_per_w // QUAD)
        n_chunks = b_per_w // CHUNK

        def offq(g):
            return pl.multiple_of(baseq + g * GROUP, GROUP)

        def issue_a(g, b):
            pltpu.async_copy(x_hbm.at[pl.ds(offq(g), GROUP)],
                             idx_v[b], sem_a[b])

        def wait_a(b):
            pltpu.make_async_copy(x_hbm.at[pl.ds(0, GROUP)],
                                  idx_v[b], sem_a[b]).wait()

        rep_off = wid * ((NREP // nw) * QROWS)

        def pack(b):
            for j in range(GROUP // 16):
                pidx_v[b][pl.ds(j * 16, 16)] = \
                    idx_v[b][pl.ds(j * 16, 16)] + rep_off

        def issue_g(b):
            pltpu.async_copy(qt_hbm.at[pidx_v[b]], rows_v[b], sem_g[b])

        def wait_g(b):
            pltpu.make_async_copy(qt_hbm.at[pidx_v[b]],
                                  rows_v[b], sem_g[b]).wait()

        def issue_w(g, b):
            pltpu.async_copy(rows_v[b],
                             out_hbm.at[pl.ds(offq(g), GROUP)], sem_w[b])

        def wait_w(b):
            pltpu.make_async_copy(rows_v[b],
                                  out_hbm.at[pl.ds(0, GROUP)],
                                  sem_w[b]).wait()

        issue_a(0, 0)
        wait_a(0)
        pack(0)
        issue_g(0)
        issue_a(1, 1)
        wait_a(1)
        pack(1)
        issue_g(1)
        issue_a(2, 2)
        wait_a(2)
        pack(2)
        issue_g(2)
        issue_a(3, 0)
        wait_g(0)
        issue_w(0, 0)

        def outer(i, carry):
            for b in range(NBUF):
                g = i * NBUF + b
                wait_a(b)
                pack(b)
                wait_w(b)
                issue_g(b)
                issue_a(g + 1, (b + 1) % NBUF)
                bw = (b + 1) % NBUF
                wait_g(bw)
                issue_w(g - 2, bw)
            return carry

        lax.fori_loop(1, (n_chunks - 2) // NBUF, outer, 0)

        n = n_chunks
        wait_a(0)
        pack(0)
        wait_w(0)
        issue_g(0)
        issue_a(n - 1, 1)
        wait_g(1)
        issue_w(n - 4, 1)

        wait_a(1)
        pack(1)
        wait_w(1)
        issue_g(1)
        wait_g(2)
        issue_w(n - 3, 2)

        wait_g(0)
        issue_w(n - 2, 0)
        wait_g(1)
        issue_w(n - 1, 1)
        wait_w(2)
        wait_w(0)
        wait_w(1)

    return lookup


def kernel(x, seg_table):
    r, c = x.shape
    B = r * c
    xi = x.astype(jnp.int32)
    quads = (xi[:, 0::4] * 27 + xi[:, 1::4] * 9
             + xi[:, 2::4] * 3 + xi[:, 3::4])
    xb = quads.reshape(B // QUAD)
    qt = _make_quad_table(seg_table)
    out = _make_sc_lookup(B)(xb, qt)
    return out.reshape(r, c, EMBED)


# speedup vs baseline: 1.3506x; 1.0216x over previous
import functools

import jax
import jax.numpy as jnp
from jax import lax
from jax.experimental import pallas as pl
from jax.experimental.pallas import tpu as pltpu
from jax.experimental.pallas import tpu_sc as plsc

EMBED = 64
QUAD = 4
QROWS = 96
QCOL = QUAD * EMBED
GROUP = 128
CHUNK = GROUP * QUAD
NBUF = 3


def _quad_table_body(t_ref, o_ref):
    t = t_ref[...]
    r = lax.broadcasted_iota(jnp.int32, (QROWS, EMBED), 0)
    rows = [jnp.broadcast_to(t[k:k + 1, :], (QROWS, EMBED)) for k in range(3)]
    parts = []
    for k in range(QUAD):
        digit = (r // (3 ** (QUAD - 1 - k))) % 3
        parts.append(jnp.where(digit == 0, rows[0],
                               jnp.where(digit == 1, rows[1], rows[2])))
    o_ref[...] = jnp.concatenate(parts, axis=1)


NREP = 32


def _make_quad_table(seg_table):
    return pl.pallas_call(
        _quad_table_body,
        grid=(NREP,),
        in_specs=[pl.BlockSpec((3, EMBED), lambda r: (0, 0))],
        out_specs=pl.BlockSpec((QROWS, QCOL), lambda r: (r, 0)),
        out_shape=jax.ShapeDtypeStruct((NREP * QROWS, QCOL), jnp.float32),
    )(seg_table)


@functools.cache
def _make_sc_lookup(B: int):
    info = plsc.get_sparse_core_info()
    nw = info.num_cores * info.num_subcores
    b_per_w = B // nw
    assert B % nw == 0 and b_per_w % CHUNK == 0
    assert (b_per_w // CHUNK - 2) % NBUF == 0 and b_per_w // CHUNK >= 5
    mesh = plsc.VectorSubcoreMesh(core_axis_name="c", subcore_axis_name="s")

    @functools.partial(
        pl.kernel,
        mesh=mesh,
        out_type=jax.ShapeDtypeStruct((B // QUAD, QCOL), jnp.float32),
        scratch_types=[
            [pltpu.VMEM((GROUP,), jnp.int32) for _ in range(NBUF)],
            [pltpu.VMEM((GROUP,), jnp.int32) for _ in range(NBUF)],
            [pltpu.VMEM((GROUP, QCOL), jnp.float32) for _ in range(NBUF)],
            [pltpu.SemaphoreType.DMA for _ in range(NBUF)],
            [pltpu.SemaphoreType.DMA for _ in range(NBUF)],
            [pltpu.SemaphoreType.DMA for _ in range(NBUF)],
        ],
    )
    def lookup(x_hbm, qt_hbm, out_hbm, idx_v, pidx_v, rows_v,
               sem_a, sem_g, sem_w):
        wid = lax.axis_index("s") * info.num_cores + lax.axis_index("c")
        baseq = wid * (b_per_w // QUAD)
        n_chunks = b_per_w // CHUNK

        def offq(g):
            return pl.multiple_of(baseq + g * GROUP, GROUP)

        def issue_a(g, b):
            pltpu.async_copy(x_hbm.at[pl.ds(offq(g), GROUP)],
                             idx_v[b], sem_a[b])

        def wait_a(b):
            pltpu.make_async_copy(x_hbm.at[pl.ds(0, GROUP)],
                                  idx_v[b], sem_a[b]).wait()

        rep_off = wid * QROWS

        def pack(b):
            for j in range(GROUP // 16):
                pidx_v[b][pl.ds(j * 16, 16)] = \
                    idx_v[b][pl.ds(j * 16, 16)] + rep_off

        def issue_g(b):
            pltpu.async_copy(qt_hbm.at[pidx_v[b]], rows_v[b], sem_g[b])

        def wait_g(b):
            pltpu.make_async_copy(qt_hbm.at[pidx_v[b]],
                                  rows_v[b], sem_g[b]).wait()

        def issue_w(g, b):
            pltpu.async_copy(rows_v[b],
                             out_hbm.at[pl.ds(offq(g), GROUP)], sem_w[b])

        def wait_w(b):
            pltpu.make_async_copy(rows_v[b],
                                  out_hbm.at[pl.ds(0, GROUP)],
                                  sem_w[b]).wait()

        issue_a(0, 0)
        wait_a(0)
        pack(0)
        issue_g(0)
        issue_a(1, 1)
        wait_a(1)
        pack(1)
        issue_g(1)
        issue_a(2, 2)
        wait_a(2)
        pack(2)
        issue_g(2)
        issue_a(3, 0)
        wait_g(0)
        issue_w(0, 0)

        def outer(i, carry):
            for b in range(NBUF):
                g = i * NBUF + b
                wait_a(b)
                pack(b)
                wait_w(b)
                issue_g(b)
                issue_a(g + 1, (b + 1) % NBUF)
                bw = (b + 1) % NBUF
                wait_g(bw)
                issue_w(g - 2, bw)
            return carry

        lax.fori_loop(1, (n_chunks - 2) // NBUF, outer, 0)

        n = n_chunks
        wait_a(0)
        pack(0)
        wait_w(0)
        issue_g(0)
        issue_a(n - 1, 1)
        wait_g(1)
        issue_w(n - 4, 1)

        wait_a(1)
        pack(1)
        wait_w(1)
        issue_g(1)
        wait_g(2)
        issue_w(n - 3, 2)

        wait_g(0)
        issue_w(n - 2, 0)
        wait_g(1)
        issue_w(n - 1, 1)
        wait_w(2)
        wait_w(0)
        wait_w(1)

    return lookup


def kernel(x, seg_table):
    r, c = x.shape
    B = r * c
    xi = x.astype(jnp.int32)
    quads = (xi[:, 0::4] * 27 + xi[:, 1::4] * 9
             + xi[:, 2::4] * 3 + xi[:, 3::4])
    xb = quads.reshape(B // QUAD)
    qt = _make_quad_table(seg_table)
    out = _make_sc_lookup(B)(xb, qt)
    return out.reshape(r, c, EMBED)
